# baseline (speedup 1.0000x reference)
import functools

import jax
import jax.numpy as jnp
import numpy as np
from jax import lax
from jax.experimental import pallas as pl
from jax.experimental.pallas import tpu as pltpu
from jax.experimental.pallas import tpu_sc as plsc

_N = 10000
_E = 50000
_H = 32
_W48 = 48
_BQ = 64.0

_NW = 32
_CHUNK = 128
_NCHUNK = -(-_E // (_NW * _CHUNK))
_EPT = _NCHUNK * _CHUNK
_RPT = 632
_ROWS = 16 * _RPT

_BN = 2000

def _rotl32(x, r):
    return ((x << np.uint32(r)) | (x >> np.uint32(32 - r))).astype(np.uint32)


def _threefry2x32(k0, k1, x0, x1):
    ks = [np.uint32(k0), np.uint32(k1),
          np.uint32(0x1BD11BDA) ^ np.uint32(k0) ^ np.uint32(k1)]
    rot = [[13, 15, 26, 6], [17, 29, 16, 24]]
    x0 = (x0 + ks[0]).astype(np.uint32)
    x1 = (x1 + ks[1]).astype(np.uint32)
    for i in range(5):
        for r in rot[i % 2]:
            x0 = (x0 + x1).astype(np.uint32)
            x1 = _rotl32(x1, r) ^ x0
        x0 = (x0 + ks[(i + 1) % 3]).astype(np.uint32)
        x1 = (x1 + ks[(i + 2) % 3] + np.uint32(i + 1)).astype(np.uint32)
    return x0, x1


def _np_uniform(fold):
    k0, k1 = _threefry2x32(0, 42, np.zeros(1, np.uint32),
                           np.full(1, fold, np.uint32))
    n = _N * _H
    a, b = _threefry2x32(k0[0], k1[0], np.zeros(n, np.uint32),
                         np.arange(n, dtype=np.uint32))
    bits = a ^ b
    f = ((bits >> np.uint32(9)) | np.uint32(0x3F800000)).view(np.float32)
    f = f - np.float32(1.0)
    return np.maximum(np.float32(0.0), f).reshape(_N, _H)


_U1 = _np_uniform(0)
_U2 = _np_uniform(1)


def _mm1_body(xt_ref, wl_ref, wr_ref, yw_ref, z_ref):
    xt = xt_ref[...]
    dn = (((0,), (0,)), ((), ()))
    y = lax.dot_general(xt, wl_ref[...], dimension_numbers=dn,
                        preferred_element_type=jnp.float32,
                        precision=lax.Precision.HIGHEST)
    z = lax.dot_general(xt, wr_ref[...], dimension_numbers=dn,
                        preferred_element_type=jnp.float32)
    col = lax.broadcasted_iota(jnp.int32, (y.shape[0], _W48 - _H), 1)
    ones_col = jnp.where(col == 0, 1.0, 0.0)
    yw_ref[...] = jnp.concatenate([y, ones_col], axis=1)
    z_ref[...] = z


def _mm1(xt, wl1, wr1):
    d_in = xt.shape[0]
    bm = 1024
    return pl.pallas_call(
        _mm1_body,
        grid=(-(-_N // bm),),
        in_specs=[
            pl.BlockSpec((d_in, bm), lambda i: (0, i)),
            pl.BlockSpec((d_in, _H), lambda i: (0, 0)),
            pl.BlockSpec((d_in, _H), lambda i: (0, 0)),
        ],
        out_specs=[
            pl.BlockSpec((bm, _W48), lambda i: (i, 0)),
            pl.BlockSpec((bm, _H), lambda i: (i, 0)),
        ],
        out_shape=[
            jax.ShapeDtypeStruct((_N, _W48), jnp.float32),
            jax.ShapeDtypeStruct((_N, _H), jnp.float32),
        ],
    )(xt, wl1, wr1)


def _seg_sum_sc(yw, src_c, dst_c, zeros_blk):
    mesh = plsc.VectorSubcoreMesh(core_axis_name="c", subcore_axis_name="s")

    @functools.partial(
        pl.kernel,
        mesh=mesh,
        compiler_params=pltpu.CompilerParams(use_tc_tiling_on_sc=False),
        out_type=jax.ShapeDtypeStruct((2, _N, _W48), jnp.float32),
        scratch_types=[
            pltpu.VMEM((_NCHUNK, _CHUNK), jnp.int32),
            pltpu.VMEM((_NCHUNK, _CHUNK), jnp.int32),
            pltpu.VMEM((_NCHUNK, _CHUNK, _W48), jnp.float32),
            pltpu.VMEM_SHARED((_ROWS, _W48), jnp.float32),
            pltpu.SemaphoreType.DMA,
            pltpu.SemaphoreType.DMA,
        ],
    )
    def k(yw_hbm, src_hbm, dst_hbm, zero_hbm, out_hbm,
          src_v, dst_v, rows_v, acc, gsem, ssem):
        cid = lax.axis_index("c")
        sid = lax.axis_index("s")
        wid = cid * 16 + sid
        pltpu.sync_copy(src_hbm.at[wid], src_v)
        pltpu.sync_copy(dst_hbm.at[wid], dst_v)
        gathers = [
            pltpu.async_copy(yw_hbm.at[src_v.at[j]], rows_v.at[j], gsem)
            for j in range(_NCHUNK)
        ]
        pltpu.sync_copy(zero_hbm, acc.at[pl.ds(sid * _RPT, _RPT)])
        plsc.subcore_barrier()
        for g in gathers:
            g.wait()
        scatters = [
            pltpu.async_copy(rows_v.at[j], acc.at[dst_v.at[j]], ssem, add=True)
            for j in range(_NCHUNK)
        ]
        for s in scatters:
            s.wait()
        plsc.subcore_barrier()
        @pl.when(sid < 15)
        def _():
            pltpu.sync_copy(acc.at[pl.ds(sid * _RPT, _RPT)],
                            out_hbm.at[cid, pl.ds(sid * _RPT, _RPT)])

        @pl.when(sid == 15)
        def _():
            rem = _N - 15 * _RPT
            pltpu.sync_copy(acc.at[pl.ds(15 * _RPT, rem)],
                            out_hbm.at[cid, pl.ds(15 * _RPT, rem)])

    return k(yw, src_c, dst_c, zeros_blk)


def _quant(t, u):
    mn = jnp.min(t, axis=1, keepdims=True)
    mx = jnp.max(t, axis=1, keepdims=True)
    xs = _BQ * (t - mn) / (mx - mn)
    a = jnp.floor(xs)
    return a + (xs - a > u).astype(jnp.float32)


def _combine1_body(p_ref, z_ref, u_ref, b_ref, hw_ref, h_ref):
    p = p_ref[0] + p_ref[1]
    agg = p[:, :_H]
    deg = p[:, _H:_H + 1]
    t = agg / jnp.maximum(deg, 1.0) + z_ref[...] + b_ref[...]
    q = _quant(t, u_ref[...])
    col = lax.broadcasted_iota(jnp.int32, (q.shape[0], _W48 - _H), 1)
    hw_ref[...] = jnp.concatenate([q, jnp.where(col == 0, 1.0, 0.0)], axis=1)
    h_ref[...] = q


def _combine1(p, z1, u1, b1):
    return pl.pallas_call(
        _combine1_body,
        grid=(_N // _BN,),
        in_specs=[
            pl.BlockSpec((2, _BN, _W48), lambda i: (0, i, 0)),
            pl.BlockSpec((_BN, _H), lambda i: (i, 0)),
            pl.BlockSpec((_BN, _H), lambda i: (i, 0)),
            pl.BlockSpec((1, _H), lambda i: (0, 0)),
        ],
        out_specs=[
            pl.BlockSpec((_BN, _W48), lambda i: (i, 0)),
            pl.BlockSpec((_BN, _H), lambda i: (i, 0)),
        ],
        out_shape=[
            jax.ShapeDtypeStruct((_N, _W48), jnp.float32),
            jax.ShapeDtypeStruct((_N, _H), jnp.float32),
        ],
    )(p, z1, u1, b1.reshape(1, _H))


def _final_body(p_ref, h_ref, u_ref, b_ref, wl_ref, wr_ref, pw1_ref, pb1_ref,
                pw2_ref, pb2_ref, out_ref):
    p = p_ref[0] + p_ref[1]
    agg = p[:, :_H]
    deg = p[:, _H:_H + 1]
    m2 = agg / jnp.maximum(deg, 1.0)
    h = h_ref[...]
    t = (jnp.dot(m2, wl_ref[...], preferred_element_type=jnp.float32)
         + jnp.dot(h, wr_ref[...], preferred_element_type=jnp.float32)
         + b_ref[...])
    q = _quant(t, u_ref[...])
    hp = jnp.dot(q, pw1_ref[...], preferred_element_type=jnp.float32)
    hp = hp + pb1_ref[...]
    logits = jnp.dot(hp, pw2_ref[...], preferred_element_type=jnp.float32)
    logits = logits + pb2_ref[...]
    m = jnp.max(logits, axis=1, keepdims=True)
    e = jnp.exp(logits - m)
    out_ref[...] = logits - m - jnp.log(jnp.sum(e, axis=1, keepdims=True))


def _final(p, h1, u2, b2, wl2, wr2, pw1, pb1, pw2, pb2):
    out_dim = pw2.shape[1]
    return pl.pallas_call(
        _final_body,
        grid=(_N // _BN,),
        in_specs=[
            pl.BlockSpec((2, _BN, _W48), lambda i: (0, i, 0)),
            pl.BlockSpec((_BN, _H), lambda i: (i, 0)),
            pl.BlockSpec((_BN, _H), lambda i: (i, 0)),
            pl.BlockSpec((1, _H), lambda i: (0, 0)),
            pl.BlockSpec((_H, _H), lambda i: (0, 0)),
            pl.BlockSpec((_H, _H), lambda i: (0, 0)),
            pl.BlockSpec((_H, _H), lambda i: (0, 0)),
            pl.BlockSpec((1, _H), lambda i: (0, 0)),
            pl.BlockSpec((_H, out_dim), lambda i: (0, 0)),
            pl.BlockSpec((1, out_dim), lambda i: (0, 0)),
        ],
        out_specs=pl.BlockSpec((_BN, out_dim), lambda i: (i, 0)),
        out_shape=jax.ShapeDtypeStruct((_N, out_dim), jnp.float32),
    )(p, h1, u2, b2.reshape(1, _H), wl2, wr2, pw1, pb1.reshape(1, _H),
      pw2, pb2.reshape(1, out_dim))


def kernel(x, edge_index, wl1, wr1, b1, wl2, wr2, b2, pw1, pb1, pw2, pb2):
    src = edge_index[0]
    dst = edge_index[1]
    epad = _NW * _EPT
    fill_s = jnp.arange(epad, dtype=jnp.int32) % _N
    src_c = fill_s.at[:_E].set(src)
    src_c = src_c.reshape(_EPT, _NW).T.reshape(_NW, _NCHUNK, _CHUNK)
    fill = _N + jnp.arange(epad, dtype=jnp.int32) % (_ROWS - _N)
    dst_c = fill.at[:_E].set(dst)
    dst_c = dst_c.reshape(_EPT, _NW).T.reshape(_NW, _NCHUNK, _CHUNK)
    zeros_blk = jnp.zeros((_RPT, _W48), jnp.float32)
    u1 = jnp.asarray(_U1)
    u2 = jnp.asarray(_U2)

    yw1, z1 = _mm1(x.T, wl1, wr1)
    p1 = _seg_sum_sc(yw1, src_c, dst_c, zeros_blk)
    h1w, h1 = _combine1(p1, z1, u1, b1)
    p2 = _seg_sum_sc(h1w, src_c, dst_c, zeros_blk)
    return _final(p2, h1, u2, b2, wl2, wr2, pw1, pb1, pw2, pb2)


# speedup vs baseline: 1.1639x; 1.1639x over previous
import functools

import jax
import jax.numpy as jnp
import numpy as np
from jax import lax
from jax.experimental import pallas as pl
from jax.experimental.pallas import tpu as pltpu
from jax.experimental.pallas import tpu_sc as plsc

_N = 10000
_E = 50000
_H = 32
_W48 = 48
_BQ = 64.0

_NW = 32
_CHUNK = 128
_NCHUNK = -(-_E // (_NW * _CHUNK))
_EPT = _NCHUNK * _CHUNK
_RPT = 632
_ROWS = 16 * _RPT

_BN = 2000

def _rotl32(x, r):
    return ((x << np.uint32(r)) | (x >> np.uint32(32 - r))).astype(np.uint32)


def _threefry2x32(k0, k1, x0, x1):
    ks = [np.uint32(k0), np.uint32(k1),
          np.uint32(0x1BD11BDA) ^ np.uint32(k0) ^ np.uint32(k1)]
    rot = [[13, 15, 26, 6], [17, 29, 16, 24]]
    x0 = (x0 + ks[0]).astype(np.uint32)
    x1 = (x1 + ks[1]).astype(np.uint32)
    for i in range(5):
        for r in rot[i % 2]:
            x0 = (x0 + x1).astype(np.uint32)
            x1 = _rotl32(x1, r) ^ x0
        x0 = (x0 + ks[(i + 1) % 3]).astype(np.uint32)
        x1 = (x1 + ks[(i + 2) % 3] + np.uint32(i + 1)).astype(np.uint32)
    return x0, x1


def _np_uniform(fold):
    k0, k1 = _threefry2x32(0, 42, np.zeros(1, np.uint32),
                           np.full(1, fold, np.uint32))
    n = _N * _H
    a, b = _threefry2x32(k0[0], k1[0], np.zeros(n, np.uint32),
                         np.arange(n, dtype=np.uint32))
    bits = a ^ b
    f = ((bits >> np.uint32(9)) | np.uint32(0x3F800000)).view(np.float32)
    f = f - np.float32(1.0)
    return np.maximum(np.float32(0.0), f).reshape(_N, _H)


_U1 = _np_uniform(0)
_U2 = _np_uniform(1)


def _mm1_body(xt_ref, wl_ref, wr_ref, yw_ref, z_ref):
    xt = xt_ref[...]
    dn = (((0,), (0,)), ((), ()))
    wl = wl_ref[...]
    xh = xt.astype(jnp.bfloat16)
    xl = (xt - xh.astype(jnp.float32)).astype(jnp.bfloat16)
    wh = wl.astype(jnp.bfloat16)
    wlo = (wl - wh.astype(jnp.float32)).astype(jnp.bfloat16)
    y = (lax.dot_general(xh, wh, dimension_numbers=dn,
                         preferred_element_type=jnp.float32)
         + lax.dot_general(xh, wlo, dimension_numbers=dn,
                           preferred_element_type=jnp.float32)
         + lax.dot_general(xl, wh, dimension_numbers=dn,
                           preferred_element_type=jnp.float32))
    z = lax.dot_general(xt, wr_ref[...], dimension_numbers=dn,
                        preferred_element_type=jnp.float32)
    col = lax.broadcasted_iota(jnp.int32, (y.shape[0], _W48 - _H), 1)
    ones_col = jnp.where(col == 0, 1.0, 0.0)
    yw_ref[...] = jnp.concatenate([y, ones_col], axis=1)
    z_ref[...] = z


def _mm1(xt, wl1, wr1):
    d_in = xt.shape[0]
    bm = 1024
    return pl.pallas_call(
        _mm1_body,
        grid=(-(-_N // bm),),
        in_specs=[
            pl.BlockSpec((d_in, bm), lambda i: (0, i)),
            pl.BlockSpec((d_in, _H), lambda i: (0, 0)),
            pl.BlockSpec((d_in, _H), lambda i: (0, 0)),
        ],
        out_specs=[
            pl.BlockSpec((bm, _W48), lambda i: (i, 0)),
            pl.BlockSpec((bm, _H), lambda i: (i, 0)),
        ],
        out_shape=[
            jax.ShapeDtypeStruct((_N, _W48), jnp.float32),
            jax.ShapeDtypeStruct((_N, _H), jnp.float32),
        ],
    )(xt, wl1, wr1)


def _seg_sum_sc(yw, src_c, dst_c, zeros_blk):
    mesh = plsc.VectorSubcoreMesh(core_axis_name="c", subcore_axis_name="s")

    @functools.partial(
        pl.kernel,
        mesh=mesh,
        compiler_params=pltpu.CompilerParams(use_tc_tiling_on_sc=False),
        out_type=jax.ShapeDtypeStruct((2, _N, _W48), jnp.float32),
        scratch_types=[
            pltpu.VMEM((_NCHUNK, _CHUNK), jnp.int32),
            pltpu.VMEM((_NCHUNK, _CHUNK), jnp.int32),
            pltpu.VMEM((_NCHUNK, _CHUNK, _W48), jnp.float32),
            pltpu.VMEM_SHARED((_ROWS, _W48), jnp.float32),
            pltpu.SemaphoreType.DMA,
            pltpu.SemaphoreType.DMA,
        ],
    )
    def k(yw_hbm, src_hbm, dst_hbm, zero_hbm, out_hbm,
          src_v, dst_v, rows_v, acc, gsem, ssem):
        cid = lax.axis_index("c")
        sid = lax.axis_index("s")
        wid = cid * 16 + sid
        pltpu.sync_copy(src_hbm.at[wid], src_v)
        pltpu.sync_copy(dst_hbm.at[wid], dst_v)
        gathers = [
            pltpu.async_copy(yw_hbm.at[src_v.at[j]], rows_v.at[j], gsem)
            for j in range(_NCHUNK)
        ]
        pltpu.sync_copy(zero_hbm, acc.at[pl.ds(sid * _RPT, _RPT)])
        plsc.subcore_barrier()
        for g in gathers:
            g.wait()
        scatters = [
            pltpu.async_copy(rows_v.at[j], acc.at[dst_v.at[j]], ssem, add=True)
            for j in range(_NCHUNK)
        ]
        for s in scatters:
            s.wait()
        plsc.subcore_barrier()
        @pl.when(sid < 15)
        def _():
            pltpu.sync_copy(acc.at[pl.ds(sid * _RPT, _RPT)],
                            out_hbm.at[cid, pl.ds(sid * _RPT, _RPT)])

        @pl.when(sid == 15)
        def _():
            rem = _N - 15 * _RPT
            pltpu.sync_copy(acc.at[pl.ds(15 * _RPT, rem)],
                            out_hbm.at[cid, pl.ds(15 * _RPT, rem)])

    return k(yw, src_c, dst_c, zeros_blk)


def _quant(t, u):
    mn = jnp.min(t, axis=1, keepdims=True)
    mx = jnp.max(t, axis=1, keepdims=True)
    xs = _BQ * (t - mn) / (mx - mn)
    a = jnp.floor(xs)
    return a + (xs - a > u).astype(jnp.float32)


def _combine1_body(p_ref, z_ref, u_ref, b_ref, hw_ref, h_ref):
    p = p_ref[0] + p_ref[1]
    agg = p[:, :_H]
    deg = p[:, _H:_H + 1]
    t = agg / jnp.maximum(deg, 1.0) + z_ref[...] + b_ref[...]
    q = _quant(t, u_ref[...])
    col = lax.broadcasted_iota(jnp.int32, (q.shape[0], _W48 - _H), 1)
    hw_ref[...] = jnp.concatenate([q, jnp.where(col == 0, 1.0, 0.0)], axis=1)
    h_ref[...] = q


def _combine1(p, z1, u1, b1):
    return pl.pallas_call(
        _combine1_body,
        grid=(_N // _BN,),
        in_specs=[
            pl.BlockSpec((2, _BN, _W48), lambda i: (0, i, 0)),
            pl.BlockSpec((_BN, _H), lambda i: (i, 0)),
            pl.BlockSpec((_BN, _H), lambda i: (i, 0)),
            pl.BlockSpec((1, _H), lambda i: (0, 0)),
        ],
        out_specs=[
            pl.BlockSpec((_BN, _W48), lambda i: (i, 0)),
            pl.BlockSpec((_BN, _H), lambda i: (i, 0)),
        ],
        out_shape=[
            jax.ShapeDtypeStruct((_N, _W48), jnp.float32),
            jax.ShapeDtypeStruct((_N, _H), jnp.float32),
        ],
    )(p, z1, u1, b1.reshape(1, _H))


def _final_body(p_ref, h_ref, u_ref, b_ref, wl_ref, wr_ref, pw1_ref, pb1_ref,
                pw2_ref, pb2_ref, out_ref):
    p = p_ref[0] + p_ref[1]
    agg = p[:, :_H]
    deg = p[:, _H:_H + 1]
    m2 = agg / jnp.maximum(deg, 1.0)
    h = h_ref[...]
    t = (jnp.dot(m2, wl_ref[...], preferred_element_type=jnp.float32)
         + jnp.dot(h, wr_ref[...], preferred_element_type=jnp.float32)
         + b_ref[...])
    q = _quant(t, u_ref[...])
    hp = jnp.dot(q, pw1_ref[...], preferred_element_type=jnp.float32)
    hp = hp + pb1_ref[...]
    logits = jnp.dot(hp, pw2_ref[...], preferred_element_type=jnp.float32)
    logits = logits + pb2_ref[...]
    m = jnp.max(logits, axis=1, keepdims=True)
    e = jnp.exp(logits - m)
    out_ref[...] = logits - m - jnp.log(jnp.sum(e, axis=1, keepdims=True))


def _final(p, h1, u2, b2, wl2, wr2, pw1, pb1, pw2, pb2):
    out_dim = pw2.shape[1]
    return pl.pallas_call(
        _final_body,
        grid=(_N // _BN,),
        in_specs=[
            pl.BlockSpec((2, _BN, _W48), lambda i: (0, i, 0)),
            pl.BlockSpec((_BN, _H), lambda i: (i, 0)),
            pl.BlockSpec((_BN, _H), lambda i: (i, 0)),
            pl.BlockSpec((1, _H), lambda i: (0, 0)),
            pl.BlockSpec((_H, _H), lambda i: (0, 0)),
            pl.BlockSpec((_H, _H), lambda i: (0, 0)),
            pl.BlockSpec((_H, _H), lambda i: (0, 0)),
            pl.BlockSpec((1, _H), lambda i: (0, 0)),
            pl.BlockSpec((_H, out_dim), lambda i: (0, 0)),
            pl.BlockSpec((1, out_dim), lambda i: (0, 0)),
        ],
        out_specs=pl.BlockSpec((_BN, out_dim), lambda i: (i, 0)),
        out_shape=jax.ShapeDtypeStruct((_N, out_dim), jnp.float32),
    )(p, h1, u2, b2.reshape(1, _H), wl2, wr2, pw1, pb1.reshape(1, _H),
      pw2, pb2.reshape(1, out_dim))


def kernel(x, edge_index, wl1, wr1, b1, wl2, wr2, b2, pw1, pb1, pw2, pb2):
    src = edge_index[0]
    dst = edge_index[1]
    epad = _NW * _EPT
    fill_s = jnp.arange(epad, dtype=jnp.int32) % _N
    src_c = fill_s.at[:_E].set(src)
    src_c = src_c.reshape(_EPT, _NW).T.reshape(_NW, _NCHUNK, _CHUNK)
    fill = _N + jnp.arange(epad, dtype=jnp.int32) % (_ROWS - _N)
    dst_c = fill.at[:_E].set(dst)
    dst_c = dst_c.reshape(_EPT, _NW).T.reshape(_NW, _NCHUNK, _CHUNK)
    zeros_blk = jnp.zeros((_RPT, _W48), jnp.float32)
    u1 = jnp.asarray(_U1)
    u2 = jnp.asarray(_U2)

    yw1, z1 = _mm1(x.T, wl1, wr1)
    p1 = _seg_sum_sc(yw1, src_c, dst_c, zeros_blk)
    h1w, h1 = _combine1(p1, z1, u1, b1)
    p2 = _seg_sum_sc(h1w, src_c, dst_c, zeros_blk)
    return _final(p2, h1, u2, b2, wl2, wr2, pw1, pb1, pw2, pb2)


# speedup vs baseline: 1.2321x; 1.0585x over previous
import functools

import jax
import jax.numpy as jnp
import numpy as np
from jax import lax
from jax.experimental import pallas as pl
from jax.experimental.pallas import tpu as pltpu
from jax.experimental.pallas import tpu_sc as plsc

_N = 10000
_E = 50000
_H = 32
_W48 = 48
_BQ = 64.0

_NW = 32
_CHUNK = 128
_NCHUNK = -(-_E // (_NW * _CHUNK))
_EPT = _NCHUNK * _CHUNK
_RPT = 632
_ROWS = 16 * _RPT

_BN = 2000

def _rotl32(x, r):
    return ((x << np.uint32(r)) | (x >> np.uint32(32 - r))).astype(np.uint32)


def _threefry2x32(k0, k1, x0, x1):
    ks = [np.uint32(k0), np.uint32(k1),
          np.uint32(0x1BD11BDA) ^ np.uint32(k0) ^ np.uint32(k1)]
    rot = [[13, 15, 26, 6], [17, 29, 16, 24]]
    x0 = (x0 + ks[0]).astype(np.uint32)
    x1 = (x1 + ks[1]).astype(np.uint32)
    for i in range(5):
        for r in rot[i % 2]:
            x0 = (x0 + x1).astype(np.uint32)
            x1 = _rotl32(x1, r) ^ x0
        x0 = (x0 + ks[(i + 1) % 3]).astype(np.uint32)
        x1 = (x1 + ks[(i + 2) % 3] + np.uint32(i + 1)).astype(np.uint32)
    return x0, x1


def _np_uniform(fold):
    k0, k1 = _threefry2x32(0, 42, np.zeros(1, np.uint32),
                           np.full(1, fold, np.uint32))
    n = _N * _H
    a, b = _threefry2x32(k0[0], k1[0], np.zeros(n, np.uint32),
                         np.arange(n, dtype=np.uint32))
    bits = a ^ b
    f = ((bits >> np.uint32(9)) | np.uint32(0x3F800000)).view(np.float32)
    f = f - np.float32(1.0)
    return np.maximum(np.float32(0.0), f).reshape(_N, _H)


_U1 = _np_uniform(0)
_U2 = _np_uniform(1)


def _mm1_body(xt_ref, w3_ref, yw_ref, z_ref):
    xt = xt_ref[...]
    bm = xt.shape[1]
    xh = xt.astype(jnp.bfloat16)
    xl = (xt - xh.astype(jnp.float32)).astype(jnp.bfloat16)
    xcat = jnp.concatenate([xh, xl], axis=1)
    p = lax.dot_general(xcat, w3_ref[...], dimension_numbers=(((0,), (0,)), ((), ())),
                        preferred_element_type=jnp.float32)
    y = p[:bm, 0:_H] + p[:bm, _H:2 * _H] + p[bm:, 0:_H]
    z = p[:bm, 2 * _H:3 * _H]
    col = lax.broadcasted_iota(jnp.int32, (y.shape[0], _W48 - _H), 1)
    ones_col = jnp.where(col == 0, 1.0, 0.0)
    yw_ref[...] = jnp.concatenate([y, ones_col], axis=1)
    z_ref[...] = z


def _mm1(xt, w3):
    d_in = xt.shape[0]
    bm = 1024
    return pl.pallas_call(
        _mm1_body,
        grid=(-(-_N // bm),),
        in_specs=[
            pl.BlockSpec((d_in, bm), lambda i: (0, i)),
            pl.BlockSpec((d_in, 3 * _H), lambda i: (0, 0)),
        ],
        out_specs=[
            pl.BlockSpec((bm, _W48), lambda i: (i, 0)),
            pl.BlockSpec((bm, _H), lambda i: (i, 0)),
        ],
        out_shape=[
            jax.ShapeDtypeStruct((_N, _W48), jnp.float32),
            jax.ShapeDtypeStruct((_N, _H), jnp.float32),
        ],
    )(xt, w3)


def _seg_sum_sc(yw, src_c, dst_c, zeros_blk):
    mesh = plsc.VectorSubcoreMesh(core_axis_name="c", subcore_axis_name="s")

    @functools.partial(
        pl.kernel,
        mesh=mesh,
        compiler_params=pltpu.CompilerParams(use_tc_tiling_on_sc=False),
        out_type=jax.ShapeDtypeStruct((2, _N, _W48), jnp.float32),
        scratch_types=[
            pltpu.VMEM((_NCHUNK, _CHUNK), jnp.int32),
            pltpu.VMEM((_NCHUNK, _CHUNK), jnp.int32),
            pltpu.VMEM((_NCHUNK, _CHUNK, _W48), jnp.float32),
            pltpu.VMEM_SHARED((_ROWS, _W48), jnp.float32),
            pltpu.SemaphoreType.DMA,
            pltpu.SemaphoreType.DMA,
        ],
    )
    def k(yw_hbm, src_hbm, dst_hbm, zero_hbm, out_hbm,
          src_v, dst_v, rows_v, acc, gsem, ssem):
        cid = lax.axis_index("c")
        sid = lax.axis_index("s")
        wid = cid * 16 + sid
        pltpu.sync_copy(src_hbm.at[wid], src_v)
        pltpu.sync_copy(dst_hbm.at[wid], dst_v)
        gathers = [
            pltpu.async_copy(yw_hbm.at[src_v.at[j]], rows_v.at[j], gsem)
            for j in range(_NCHUNK)
        ]
        pltpu.sync_copy(zero_hbm, acc.at[pl.ds(sid * _RPT, _RPT)])
        plsc.subcore_barrier()
        for g in gathers:
            g.wait()
        scatters = [
            pltpu.async_copy(rows_v.at[j], acc.at[dst_v.at[j]], ssem, add=True)
            for j in range(_NCHUNK)
        ]
        for s in scatters:
            s.wait()
        plsc.subcore_barrier()
        @pl.when(sid < 15)
        def _():
            pltpu.sync_copy(acc.at[pl.ds(sid * _RPT, _RPT)],
                            out_hbm.at[cid, pl.ds(sid * _RPT, _RPT)])

        @pl.when(sid == 15)
        def _():
            rem = _N - 15 * _RPT
            pltpu.sync_copy(acc.at[pl.ds(15 * _RPT, rem)],
                            out_hbm.at[cid, pl.ds(15 * _RPT, rem)])

    return k(yw, src_c, dst_c, zeros_blk)


def _quant(t, u):
    mn = jnp.min(t, axis=1, keepdims=True)
    mx = jnp.max(t, axis=1, keepdims=True)
    xs = _BQ * (t - mn) / (mx - mn)
    a = jnp.floor(xs)
    return a + (xs - a > u).astype(jnp.float32)


def _combine1_body(p_ref, z_ref, u_ref, b_ref, hw_ref, h_ref):
    p = p_ref[0] + p_ref[1]
    agg = p[:, :_H]
    deg = p[:, _H:_H + 1]
    t = agg / jnp.maximum(deg, 1.0) + z_ref[...] + b_ref[...]
    q = _quant(t, u_ref[...])
    col = lax.broadcasted_iota(jnp.int32, (q.shape[0], _W48 - _H), 1)
    hw_ref[...] = jnp.concatenate([q, jnp.where(col == 0, 1.0, 0.0)], axis=1)
    h_ref[...] = q


def _combine1(p, z1, u1, b1):
    return pl.pallas_call(
        _combine1_body,
        grid=(_N // _BN,),
        in_specs=[
            pl.BlockSpec((2, _BN, _W48), lambda i: (0, i, 0)),
            pl.BlockSpec((_BN, _H), lambda i: (i, 0)),
            pl.BlockSpec((_BN, _H), lambda i: (i, 0)),
            pl.BlockSpec((1, _H), lambda i: (0, 0)),
        ],
        out_specs=[
            pl.BlockSpec((_BN, _W48), lambda i: (i, 0)),
            pl.BlockSpec((_BN, _H), lambda i: (i, 0)),
        ],
        out_shape=[
            jax.ShapeDtypeStruct((_N, _W48), jnp.float32),
            jax.ShapeDtypeStruct((_N, _H), jnp.float32),
        ],
    )(p, z1, u1, b1.reshape(1, _H))


def _final_body(p_ref, h_ref, u_ref, b_ref, wl_ref, wr_ref, pw1_ref, pb1_ref,
                pw2_ref, pb2_ref, out_ref):
    p = p_ref[0] + p_ref[1]
    agg = p[:, :_H]
    deg = p[:, _H:_H + 1]
    m2 = agg / jnp.maximum(deg, 1.0)
    h = h_ref[...]
    t = (jnp.dot(m2, wl_ref[...], preferred_element_type=jnp.float32)
         + jnp.dot(h, wr_ref[...], preferred_element_type=jnp.float32)
         + b_ref[...])
    q = _quant(t, u_ref[...])
    hp = jnp.dot(q, pw1_ref[...], preferred_element_type=jnp.float32)
    hp = hp + pb1_ref[...]
    logits = jnp.dot(hp, pw2_ref[...], preferred_element_type=jnp.float32)
    logits = logits + pb2_ref[...]
    m = jnp.max(logits, axis=1, keepdims=True)
    e = jnp.exp(logits - m)
    out_ref[...] = logits - m - jnp.log(jnp.sum(e, axis=1, keepdims=True))


def _final(p, h1, u2, b2, wl2, wr2, pw1, pb1, pw2, pb2):
    out_dim = pw2.shape[1]
    return pl.pallas_call(
        _final_body,
        grid=(_N // _BN,),
        in_specs=[
            pl.BlockSpec((2, _BN, _W48), lambda i: (0, i, 0)),
            pl.BlockSpec((_BN, _H), lambda i: (i, 0)),
            pl.BlockSpec((_BN, _H), lambda i: (i, 0)),
            pl.BlockSpec((1, _H), lambda i: (0, 0)),
            pl.BlockSpec((_H, _H), lambda i: (0, 0)),
            pl.BlockSpec((_H, _H), lambda i: (0, 0)),
            pl.BlockSpec((_H, _H), lambda i: (0, 0)),
            pl.BlockSpec((1, _H), lambda i: (0, 0)),
            pl.BlockSpec((_H, out_dim), lambda i: (0, 0)),
            pl.BlockSpec((1, out_dim), lambda i: (0, 0)),
        ],
        out_specs=pl.BlockSpec((_BN, out_dim), lambda i: (i, 0)),
        out_shape=jax.ShapeDtypeStruct((_N, out_dim), jnp.float32),
    )(p, h1, u2, b2.reshape(1, _H), wl2, wr2, pw1, pb1.reshape(1, _H),
      pw2, pb2.reshape(1, out_dim))


def kernel(x, edge_index, wl1, wr1, b1, wl2, wr2, b2, pw1, pb1, pw2, pb2):
    src = edge_index[0]
    dst = edge_index[1]
    epad = _NW * _EPT
    fill_s = jnp.arange(epad, dtype=jnp.int32) % _N
    src_c = fill_s.at[:_E].set(src)
    src_c = src_c.reshape(_EPT, _NW).T.reshape(_NW, _NCHUNK, _CHUNK)
    fill = _N + jnp.arange(epad, dtype=jnp.int32) % (_ROWS - _N)
    dst_c = fill.at[:_E].set(dst)
    dst_c = dst_c.reshape(_EPT, _NW).T.reshape(_NW, _NCHUNK, _CHUNK)
    zeros_blk = jnp.zeros((_RPT, _W48), jnp.float32)
    u1 = jnp.asarray(_U1)
    u2 = jnp.asarray(_U2)

    wh = wl1.astype(jnp.bfloat16)
    wlo = (wl1 - wh.astype(jnp.float32)).astype(jnp.bfloat16)
    w3 = jnp.concatenate([wh, wlo, wr1.astype(jnp.bfloat16)], axis=1)
    yw1, z1 = _mm1(x.T, w3)
    p1 = _seg_sum_sc(yw1, src_c, dst_c, zeros_blk)
    h1w, h1 = _combine1(p1, z1, u1, b1)
    p2 = _seg_sum_sc(h1w, src_c, dst_c, zeros_blk)
    return _final(p2, h1, u2, b2, wl2, wr2, pw1, pb1, pw2, pb2)


# speedup vs baseline: 1.2690x; 1.0299x over previous
import functools

import jax
import jax.numpy as jnp
import numpy as np
from jax import lax
from jax.experimental import pallas as pl
from jax.experimental.pallas import tpu as pltpu
from jax.experimental.pallas import tpu_sc as plsc

_N = 10000
_E = 50000
_H = 32
_W48 = 48
_BQ = 64.0

_NW = 32
_CHUNK = 128
_NCHUNK = -(-_E // (_NW * _CHUNK))
_EPT = _NCHUNK * _CHUNK
_RPT = 632
_ROWS = 16 * _RPT

_BN = 2000

def _rotl32(x, r):
    return ((x << np.uint32(r)) | (x >> np.uint32(32 - r))).astype(np.uint32)


def _threefry2x32(k0, k1, x0, x1):
    ks = [np.uint32(k0), np.uint32(k1),
          np.uint32(0x1BD11BDA) ^ np.uint32(k0) ^ np.uint32(k1)]
    rot = [[13, 15, 26, 6], [17, 29, 16, 24]]
    x0 = (x0 + ks[0]).astype(np.uint32)
    x1 = (x1 + ks[1]).astype(np.uint32)
    for i in range(5):
        for r in rot[i % 2]:
            x0 = (x0 + x1).astype(np.uint32)
            x1 = _rotl32(x1, r) ^ x0
        x0 = (x0 + ks[(i + 1) % 3]).astype(np.uint32)
        x1 = (x1 + ks[(i + 2) % 3] + np.uint32(i + 1)).astype(np.uint32)
    return x0, x1


def _np_uniform(fold):
    k0, k1 = _threefry2x32(0, 42, np.zeros(1, np.uint32),
                           np.full(1, fold, np.uint32))
    n = _N * _H
    a, b = _threefry2x32(k0[0], k1[0], np.zeros(n, np.uint32),
                         np.arange(n, dtype=np.uint32))
    bits = a ^ b
    f = ((bits >> np.uint32(9)) | np.uint32(0x3F800000)).view(np.float32)
    f = f - np.float32(1.0)
    return np.maximum(np.float32(0.0), f).reshape(_N, _H)


_U1 = _np_uniform(0)
_U2 = _np_uniform(1)


def _mm1_body(xt_ref, w3_ref, yw_ref, z_ref):
    xt = xt_ref[...]
    bm = xt.shape[1]
    xh = xt.astype(jnp.bfloat16)
    xl = (xt - xh.astype(jnp.float32)).astype(jnp.bfloat16)
    xcat = jnp.concatenate([xh, xl], axis=1)
    p = lax.dot_general(xcat, w3_ref[...], dimension_numbers=(((0,), (0,)), ((), ())),
                        preferred_element_type=jnp.float32)
    y = p[:bm, 0:_H] + p[:bm, _H:2 * _H] + p[bm:, 0:_H]
    z = p[:bm, 2 * _H:3 * _H]
    col = lax.broadcasted_iota(jnp.int32, (y.shape[0], _W48 - _H), 1)
    ones_col = jnp.where(col == 0, 1.0, 0.0)
    yw_ref[...] = jnp.concatenate([y, ones_col], axis=1)
    z_ref[...] = z


def _mm1(xt, w3):
    d_in = xt.shape[0]
    bm = 1024
    return pl.pallas_call(
        _mm1_body,
        grid=(-(-_N // bm),),
        in_specs=[
            pl.BlockSpec((d_in, bm), lambda i: (0, i)),
            pl.BlockSpec((d_in, 3 * _H), lambda i: (0, 0)),
        ],
        out_specs=[
            pl.BlockSpec((bm, _W48), lambda i: (i, 0)),
            pl.BlockSpec((bm, _H), lambda i: (i, 0)),
        ],
        out_shape=[
            jax.ShapeDtypeStruct((_N, _W48), jnp.float32),
            jax.ShapeDtypeStruct((_N, _H), jnp.float32),
        ],
    )(xt, w3)


def _seg_sum_sc(yw, src_c, dst_c, zeros_blk, width):
    mesh = plsc.VectorSubcoreMesh(core_axis_name="c", subcore_axis_name="s")

    @functools.partial(
        pl.kernel,
        mesh=mesh,
        compiler_params=pltpu.CompilerParams(use_tc_tiling_on_sc=False),
        out_type=jax.ShapeDtypeStruct((2, _N, width), jnp.float32),
        scratch_types=[
            pltpu.VMEM((_NCHUNK, _CHUNK), jnp.int32),
            pltpu.VMEM((_NCHUNK, _CHUNK), jnp.int32),
            pltpu.VMEM((_NCHUNK, _CHUNK, width), jnp.float32),
            pltpu.VMEM_SHARED((_ROWS, width), jnp.float32),
            pltpu.SemaphoreType.DMA,
            pltpu.SemaphoreType.DMA,
        ],
    )
    def k(yw_hbm, src_hbm, dst_hbm, zero_hbm, out_hbm,
          src_v, dst_v, rows_v, acc, gsem, ssem):
        cid = lax.axis_index("c")
        sid = lax.axis_index("s")
        wid = cid * 16 + sid
        pltpu.sync_copy(src_hbm.at[wid], src_v)
        pltpu.sync_copy(dst_hbm.at[wid], dst_v)
        gathers = [
            pltpu.async_copy(yw_hbm.at[src_v.at[j]], rows_v.at[j], gsem)
            for j in range(_NCHUNK)
        ]
        pltpu.sync_copy(zero_hbm, acc.at[pl.ds(sid * _RPT, _RPT)])
        plsc.subcore_barrier()
        for g in gathers:
            g.wait()
        scatters = [
            pltpu.async_copy(rows_v.at[j], acc.at[dst_v.at[j]], ssem, add=True)
            for j in range(_NCHUNK)
        ]
        for s in scatters:
            s.wait()
        plsc.subcore_barrier()
        @pl.when(sid < 15)
        def _():
            pltpu.sync_copy(acc.at[pl.ds(sid * _RPT, _RPT)],
                            out_hbm.at[cid, pl.ds(sid * _RPT, _RPT)])

        @pl.when(sid == 15)
        def _():
            rem = _N - 15 * _RPT
            pltpu.sync_copy(acc.at[pl.ds(15 * _RPT, rem)],
                            out_hbm.at[cid, pl.ds(15 * _RPT, rem)])

    return k(yw, src_c, dst_c, zeros_blk)


def _quant(t, u):
    mn = jnp.min(t, axis=1, keepdims=True)
    mx = jnp.max(t, axis=1, keepdims=True)
    xs = _BQ * (t - mn) / (mx - mn)
    a = jnp.floor(xs)
    return a + (xs - a > u).astype(jnp.float32)


def _combine1_body(p_ref, z_ref, u_ref, b_ref, h_ref, d_ref):
    p = p_ref[0] + p_ref[1]
    agg = p[:, :_H]
    deg = p[:, _H:_H + 1]
    degc = jnp.maximum(deg, 1.0)
    t = agg / degc + z_ref[...] + b_ref[...]
    h_ref[...] = _quant(t, u_ref[...])
    d_ref[...] = degc


def _combine1(p, z1, u1, b1):
    return pl.pallas_call(
        _combine1_body,
        grid=(_N // _BN,),
        in_specs=[
            pl.BlockSpec((2, _BN, _W48), lambda i: (0, i, 0)),
            pl.BlockSpec((_BN, _H), lambda i: (i, 0)),
            pl.BlockSpec((_BN, _H), lambda i: (i, 0)),
            pl.BlockSpec((1, _H), lambda i: (0, 0)),
        ],
        out_specs=[
            pl.BlockSpec((_BN, _H), lambda i: (i, 0)),
            pl.BlockSpec((_BN, 1), lambda i: (i, 0)),
        ],
        out_shape=[
            jax.ShapeDtypeStruct((_N, _H), jnp.float32),
            jax.ShapeDtypeStruct((_N, 1), jnp.float32),
        ],
    )(p, z1, u1, b1.reshape(1, _H))


def _final_body(p_ref, h_ref, d_ref, u_ref, b_ref, wl_ref, wr_ref, pw1_ref,
                pb1_ref, pw2_ref, pb2_ref, out_ref):
    agg = p_ref[0] + p_ref[1]
    m2 = agg / d_ref[...]
    h = h_ref[...]
    t = (jnp.dot(m2, wl_ref[...], preferred_element_type=jnp.float32)
         + jnp.dot(h, wr_ref[...], preferred_element_type=jnp.float32)
         + b_ref[...])
    q = _quant(t, u_ref[...])
    hp = jnp.dot(q, pw1_ref[...], preferred_element_type=jnp.float32)
    hp = hp + pb1_ref[...]
    logits = jnp.dot(hp, pw2_ref[...], preferred_element_type=jnp.float32)
    logits = logits + pb2_ref[...]
    m = jnp.max(logits, axis=1, keepdims=True)
    e = jnp.exp(logits - m)
    out_ref[...] = logits - m - jnp.log(jnp.sum(e, axis=1, keepdims=True))


def _final(p, h1, degc, u2, b2, wl2, wr2, pw1, pb1, pw2, pb2):
    out_dim = pw2.shape[1]
    return pl.pallas_call(
        _final_body,
        grid=(_N // _BN,),
        in_specs=[
            pl.BlockSpec((2, _BN, _H), lambda i: (0, i, 0)),
            pl.BlockSpec((_BN, _H), lambda i: (i, 0)),
            pl.BlockSpec((_BN, 1), lambda i: (i, 0)),
            pl.BlockSpec((_BN, _H), lambda i: (i, 0)),
            pl.BlockSpec((1, _H), lambda i: (0, 0)),
            pl.BlockSpec((_H, _H), lambda i: (0, 0)),
            pl.BlockSpec((_H, _H), lambda i: (0, 0)),
            pl.BlockSpec((_H, _H), lambda i: (0, 0)),
            pl.BlockSpec((1, _H), lambda i: (0, 0)),
            pl.BlockSpec((_H, out_dim), lambda i: (0, 0)),
            pl.BlockSpec((1, out_dim), lambda i: (0, 0)),
        ],
        out_specs=pl.BlockSpec((_BN, out_dim), lambda i: (i, 0)),
        out_shape=jax.ShapeDtypeStruct((_N, out_dim), jnp.float32),
    )(p, h1, degc, u2, b2.reshape(1, _H), wl2, wr2, pw1, pb1.reshape(1, _H),
      pw2, pb2.reshape(1, out_dim))


def kernel(x, edge_index, wl1, wr1, b1, wl2, wr2, b2, pw1, pb1, pw2, pb2):
    src = edge_index[0]
    dst = edge_index[1]
    epad = _NW * _EPT
    fill_s = jnp.arange(epad, dtype=jnp.int32) % _N
    src_c = fill_s.at[:_E].set(src)
    src_c = src_c.reshape(_EPT, _NW).T.reshape(_NW, _NCHUNK, _CHUNK)
    fill = _N + jnp.arange(epad, dtype=jnp.int32) % (_ROWS - _N)
    dst_c = fill.at[:_E].set(dst)
    dst_c = dst_c.reshape(_EPT, _NW).T.reshape(_NW, _NCHUNK, _CHUNK)
    zeros48 = jnp.zeros((_RPT, _W48), jnp.float32)
    zeros32 = jnp.zeros((_RPT, _H), jnp.float32)
    u1 = jnp.asarray(_U1)
    u2 = jnp.asarray(_U2)

    wh = wl1.astype(jnp.bfloat16)
    wlo = (wl1 - wh.astype(jnp.float32)).astype(jnp.bfloat16)
    w3 = jnp.concatenate([wh, wlo, wr1.astype(jnp.bfloat16)], axis=1)
    yw1, z1 = _mm1(x.T, w3)
    p1 = _seg_sum_sc(yw1, src_c, dst_c, zeros48, _W48)
    h1, degc = _combine1(p1, z1, u1, b1)
    p2 = _seg_sum_sc(h1, src_c, dst_c, zeros32, _H)
    return _final(p2, h1, degc, u2, b2, wl2, wr2, pw1, pb1, pw2, pb2)
